# baseline (speedup 1.0000x reference)
import functools
import math

import jax
import jax.numpy as jnp
from jax import lax
from jax.experimental import pallas as pl
from jax.experimental.pallas import tpu as pltpu
from jax.experimental.pallas import tpu_sc as plsc

EMB = 64
SCALE = math.sqrt(EMB)
NC, NS, L = 2, 16, 16
NW = NC * NS
G = 128


def kernel(tokens, table):
    B = tokens.shape[0] * tokens.shape[1]
    b_per_w = B // NW
    W = 1024
    steps = b_per_w // W
    assert b_per_w % W == 0

    idx = tokens.reshape(B // G, G).astype(jnp.int32)
    mesh = plsc.VectorSubcoreMesh(core_axis_name="c", subcore_axis_name="s")

    @functools.partial(
        pl.kernel,
        mesh=mesh,
        out_type=jax.ShapeDtypeStruct((B, EMB), jnp.float32),
        scratch_types=[
            pltpu.VMEM((W // G, G), jnp.int32),
            pltpu.VMEM((W, EMB), jnp.float32),
            pltpu.SemaphoreType.DMA,
        ],
        compiler_params=pltpu.CompilerParams(use_tc_tiling_on_sc=False),
    )
    def emb_kernel(idx_hbm, table_hbm, out_hbm, idx_v, rows_v, sem):
        wid = lax.axis_index("s") * NC + lax.axis_index("c")
        base = wid * b_per_w

        @pl.loop(0, steps)
        def _(i):
            off = pl.multiple_of(base + i * W, W)
            pltpu.sync_copy(idx_hbm.at[pl.ds(pl.multiple_of(off // G, 8), W // G)], idx_v)
            for j in range(W // G):
                pltpu.async_copy(
                    table_hbm.at[idx_v.at[j]],
                    rows_v.at[pl.ds(j * G, G)],
                    sem,
                )
            for j in range(W // G):
                pltpu.make_async_copy(
                    table_hbm.at[idx_v.at[j]],
                    rows_v.at[pl.ds(j * G, G)],
                    sem,
                ).wait()

            @pl.loop(0, W)
            def _(r):
                for c in range(0, EMB, L):
                    rows_v[r, pl.ds(c, L)] = rows_v[r, pl.ds(c, L)] * SCALE

            pltpu.sync_copy(rows_v, out_hbm.at[pl.ds(off, W)])

    out = emb_kernel(idx, table)
    return out.reshape(tokens.shape + (EMB,))
